# baseline (speedup 1.0000x reference)
import functools

import jax
import jax.numpy as jnp
import numpy as np
from jax import lax
from jax.experimental import pallas as pl
from jax.experimental.pallas import tpu as pltpu
from jax.experimental.pallas import tpu_sc as plsc

M = 8
N = 32768
D = 128
P0, P1 = 64, 64
G = P0 * P1

NC = 2
NS = 16
L = 16

B_PER_CORE = M // NC
PTS_PER_TILE = N // NS
CH = 128
NCH = PTS_PER_TILE // CH
ROWS_PER_TILE = G // NS

SPACING = np.float32(1.0 / 63.0)
MAGIC = np.float32(12582912.0)


def _sc_grid_mean(xt_flat, z2, latf):
    mesh = plsc.VectorSubcoreMesh(core_axis_name="c", subcore_axis_name="s")

    @functools.partial(
        pl.kernel,
        mesh=mesh,
        out_type=jax.ShapeDtypeStruct((M * G, D), jnp.float32),
        scratch_types=[
            pltpu.VMEM((CH, D), jnp.float32),
            pltpu.VMEM((CH, D), jnp.float32),
            pltpu.VMEM((CH, D), jnp.float32),
            pltpu.VMEM((2 * PTS_PER_TILE,), jnp.float32),
            pltpu.VMEM((CH,), jnp.int32),
            pltpu.SemaphoreType.DMA,
            pltpu.SemaphoreType.DMA,
            pltpu.VMEM_SHARED((G, D), jnp.float32),
            pltpu.VMEM_SHARED((G, D), jnp.float32),
        ],
    )
    def k(xt_hbm, z_hbm, lat_hbm, out_hbm,
          zbuf, cbuf, lbuf, xb, idxc, gsem, ssem, accum, counts):
        c = lax.axis_index("c")
        t = lax.axis_index("s")

        zero16 = jnp.zeros((L,), jnp.float32)
        one16 = jnp.full((L,), 1.0, jnp.float32)

        def fill(buf, val16):
            def body(r, _):
                for cc in range(D // L):
                    buf[r, pl.ds(cc * L, L)] = val16
                return _
            lax.fori_loop(0, CH, body, None)

        fill(cbuf, zero16)
        for h in range(ROWS_PER_TILE // CH):
            row0 = t * ROWS_PER_TILE + h * CH
            pltpu.sync_copy(cbuf, accum.at[pl.ds(row0, CH)])
            pltpu.sync_copy(cbuf, counts.at[pl.ds(row0, CH)])

        base = t * PTS_PER_TILE
        for kk in range(B_PER_CORE):
            b = c * B_PER_CORE + kk
            plsc.subcore_barrier()

            fill(lbuf, one16)
            pltpu.sync_copy(xt_hbm.at[pl.ds(b * N + base, PTS_PER_TILE)],
                            xb.at[pl.ds(0, PTS_PER_TILE)])
            pltpu.sync_copy(xt_hbm.at[pl.ds(M * N + b * N + base,
                                            PTS_PER_TILE)],
                            xb.at[pl.ds(PTS_PER_TILE, PTS_PER_TILE)])

            pltpu.async_copy(z_hbm.at[pl.ds(b * N + base, CH)], zbuf, gsem)

            def chunk_body(j, _):
                def group_body(g, _):
                    off = j * CH + g * L
                    v0 = xb[pl.ds(off, L)] / SPACING
                    v1 = xb[pl.ds(PTS_PER_TILE + off, L)] / SPACING
                    r0 = (v0 + MAGIC) - MAGIC
                    r1 = (v1 + MAGIC) - MAGIC
                    r0 = jnp.minimum(jnp.maximum(r0, 0.0), 63.0)
                    r1 = jnp.minimum(jnp.maximum(r1, 0.0), 63.0)
                    flat = r0.astype(jnp.int32) * P1 + r1.astype(jnp.int32)
                    flat = jnp.minimum(jnp.maximum(flat, 0), G - 1)
                    idxc[pl.ds(g * L, L)] = flat
                    return _
                lax.fori_loop(0, CH // L, group_body, None)

                pltpu.make_async_copy(z_hbm.at[pl.ds(0, CH)], zbuf,
                                      gsem).wait()
                pltpu.async_copy(zbuf, accum.at[idxc], add=True, sem=ssem)
                pltpu.sync_copy(lbuf, counts.at[idxc], add=True)
                pltpu.make_async_copy(z_hbm.at[pl.ds(0, CH)], zbuf,
                                      ssem).wait()

                @pl.when(j < NCH - 1)
                def _prefetch():
                    pltpu.async_copy(
                        z_hbm.at[pl.ds(b * N + base + (j + 1) * CH, CH)],
                        zbuf, gsem)
                return _
            lax.fori_loop(0, NCH, chunk_body, None)

            plsc.subcore_barrier()

            for h in range(ROWS_PER_TILE // CH):
                row0 = t * ROWS_PER_TILE + h * CH
                pltpu.sync_copy(accum.at[pl.ds(row0, CH)], zbuf)
                pltpu.sync_copy(counts.at[pl.ds(row0, CH)], cbuf)
                pltpu.sync_copy(lat_hbm.at[pl.ds(row0, CH)], lbuf)

                def row_body(r, _):
                    cnt = cbuf[r, pl.ds(0, L)]
                    rec = jnp.where(cnt >= 1.0,
                                    1.0 / jnp.maximum(cnt, 1.0), 0.0)
                    for cc in range(D // L):
                        s = zbuf[r, pl.ds(cc * L, L)]
                        zbuf[r, pl.ds(cc * L, L)] = (
                            s * rec + lbuf[r, pl.ds(cc * L, L)])
                    return _
                lax.fori_loop(0, CH, row_body, None)

                pltpu.sync_copy(zbuf, out_hbm.at[pl.ds(b * G + row0, CH)])
                fill(cbuf, zero16)
                pltpu.sync_copy(cbuf, accum.at[pl.ds(row0, CH)])
                pltpu.sync_copy(cbuf, counts.at[pl.ds(row0, CH)])

    return k(xt_flat, z2, latf)


def kernel(x, z, latents):
    m, n, _ = x.shape
    axes = [jnp.linspace(0.0, 1.0, p) for p in (P0, P1)]
    mesh_xy = jnp.meshgrid(*axes, indexing="ij")
    grid = jnp.stack(mesh_xy, axis=-1)
    x_grid = jnp.broadcast_to(grid[None], (m, P0, P1, 2))

    xt_flat = jnp.transpose(x, (2, 0, 1)).reshape(2 * m * n)
    z2 = z.reshape(m * n, D)
    latf = latents.reshape(G, D)
    zg = _sc_grid_mean(xt_flat, z2, latf)
    z_grid = zg.reshape(m, P0, P1, D)
    return (x_grid, z_grid)


# speedup vs baseline: 1.1809x; 1.1809x over previous
import functools

import jax
import jax.numpy as jnp
import numpy as np
from jax import lax
from jax.experimental import pallas as pl
from jax.experimental.pallas import tpu as pltpu
from jax.experimental.pallas import tpu_sc as plsc

M = 8
N = 32768
D = 128
P0, P1 = 64, 64
G = P0 * P1

NC = 2
NS = 16
L = 16

B_PER_CORE = M // NC
PTS_PER_TILE = N // NS
CH = 128
NCH = PTS_PER_TILE // CH
ROWS_PER_TILE = G // NS

SPACING = np.float32(1.0 / 63.0)
MAGIC = np.float32(12582912.0)


def _sc_grid_mean(xt_flat, z2, latf):
    mesh = plsc.VectorSubcoreMesh(core_axis_name="c", subcore_axis_name="s")

    @functools.partial(
        pl.kernel,
        mesh=mesh,
        out_type=jax.ShapeDtypeStruct((M * G, D), jnp.float32),
        scratch_types=[
            pltpu.VMEM((CH, D), jnp.float32),
            pltpu.VMEM((CH, D), jnp.float32),
            pltpu.VMEM((CH, D), jnp.float32),
            pltpu.VMEM((2 * PTS_PER_TILE,), jnp.float32),
            pltpu.VMEM((CH,), jnp.int32),
            pltpu.SemaphoreType.DMA,
            pltpu.VMEM_SHARED((G, D), jnp.float32),
            pltpu.VMEM_SHARED((G, D), jnp.float32),
        ],
    )
    def k(xt_hbm, z_hbm, lat_hbm, out_hbm,
          zbuf, cbuf, lbuf, xb, idxc, gsem, accum, counts):
        c = lax.axis_index("c")
        t = lax.axis_index("s")

        zero16 = jnp.zeros((L,), jnp.float32)
        one16 = jnp.full((L,), 1.0, jnp.float32)

        def fill(buf, val16):
            def body(r, _):
                for cc in range(D // L):
                    buf[r, pl.ds(cc * L, L)] = val16
                return _
            lax.fori_loop(0, CH, body, None)

        fill(cbuf, zero16)
        for h in range(ROWS_PER_TILE // CH):
            row0 = t * ROWS_PER_TILE + h * CH
            pltpu.sync_copy(cbuf, accum.at[pl.ds(row0, CH)])
            pltpu.sync_copy(cbuf, counts.at[pl.ds(row0, CH)])

        base = t * PTS_PER_TILE
        for kk in range(B_PER_CORE):
            b = c * B_PER_CORE + kk
            plsc.subcore_barrier()

            fill(lbuf, one16)
            pltpu.sync_copy(xt_hbm.at[pl.ds(b * N + base, PTS_PER_TILE)],
                            xb.at[pl.ds(0, PTS_PER_TILE)])
            pltpu.sync_copy(xt_hbm.at[pl.ds(M * N + b * N + base,
                                            PTS_PER_TILE)],
                            xb.at[pl.ds(PTS_PER_TILE, PTS_PER_TILE)])

            pltpu.async_copy(z_hbm.at[pl.ds(b * N + base, CH)], zbuf, gsem)

            def chunk_body(j, _):
                def group_body(g, _):
                    off = j * CH + g * L
                    v0 = xb[pl.ds(off, L)] / SPACING
                    v1 = xb[pl.ds(PTS_PER_TILE + off, L)] / SPACING
                    r0 = (v0 + MAGIC) - MAGIC
                    r1 = (v1 + MAGIC) - MAGIC
                    r0 = jnp.minimum(jnp.maximum(r0, 0.0), 63.0)
                    r1 = jnp.minimum(jnp.maximum(r1, 0.0), 63.0)
                    flat = r0.astype(jnp.int32) * P1 + r1.astype(jnp.int32)
                    flat = jnp.minimum(jnp.maximum(flat, 0), G - 1)
                    idxc[pl.ds(g * L, L)] = flat
                    return _
                lax.fori_loop(0, CH // L, group_body, None)

                pltpu.make_async_copy(z_hbm.at[pl.ds(0, CH)], zbuf,
                                      gsem).wait()
                pltpu.sync_copy(zbuf, accum.at[idxc], add=True)

                @pl.when(j < NCH - 1)
                def _prefetch():
                    pltpu.async_copy(
                        z_hbm.at[pl.ds(b * N + base + (j + 1) * CH, CH)],
                        zbuf, gsem)

                pltpu.sync_copy(lbuf, counts.at[idxc], add=True)
                return _
            lax.fori_loop(0, NCH, chunk_body, None)

            plsc.subcore_barrier()

            for h in range(ROWS_PER_TILE // CH):
                row0 = t * ROWS_PER_TILE + h * CH
                pltpu.sync_copy(accum.at[pl.ds(row0, CH)], zbuf)
                pltpu.sync_copy(counts.at[pl.ds(row0, CH)], cbuf)
                pltpu.sync_copy(lat_hbm.at[pl.ds(row0, CH)], lbuf)

                def row_body(r, _):
                    cnt = cbuf[r, pl.ds(0, L)]
                    rec = jnp.where(cnt >= 1.0,
                                    1.0 / jnp.maximum(cnt, 1.0), 0.0)
                    for cc in range(D // L):
                        s = zbuf[r, pl.ds(cc * L, L)]
                        zbuf[r, pl.ds(cc * L, L)] = (
                            s * rec + lbuf[r, pl.ds(cc * L, L)])
                    return _
                lax.fori_loop(0, CH, row_body, None)

                pltpu.sync_copy(zbuf, out_hbm.at[pl.ds(b * G + row0, CH)])
                if kk < B_PER_CORE - 1:
                    fill(cbuf, zero16)
                    pltpu.sync_copy(cbuf, accum.at[pl.ds(row0, CH)])
                    pltpu.sync_copy(cbuf, counts.at[pl.ds(row0, CH)])

    return k(xt_flat, z2, latf)


def kernel(x, z, latents):
    m, n, _ = x.shape
    axes = [jnp.linspace(0.0, 1.0, p) for p in (P0, P1)]
    mesh_xy = jnp.meshgrid(*axes, indexing="ij")
    grid = jnp.stack(mesh_xy, axis=-1)
    x_grid = jnp.broadcast_to(grid[None], (m, P0, P1, 2))

    xt_flat = jnp.transpose(x, (2, 0, 1)).reshape(2 * m * n)
    z2 = z.reshape(m * n, D)
    latf = latents.reshape(G, D)
    zg = _sc_grid_mean(xt_flat, z2, latf)
    z_grid = zg.reshape(m, P0, P1, D)
    return (x_grid, z_grid)
